# baseline (speedup 1.0000x reference)
import functools

import jax
import jax.numpy as jnp
from jax import lax
from jax.experimental import pallas as pl
from jax.experimental.pallas import tpu as pltpu
from jax.experimental.pallas import tpu_sc as plsc

L = 16
CHUNK = 128
EUNROLL = 4

_GATHER_DNUMS = lax.GatherDimensionNumbers(
    offset_dims=(), collapsed_slice_dims=(0,), start_index_map=(0,))


def _take16(x, idx2d):
    return lax.gather(x, idx2d, _GATHER_DNUMS, (1,),
                      mode=lax.GatherScatterMode.PROMISE_IN_BOUNDS)


def _make_sc_call(b, dim2):
    info = plsc.get_sparse_core_info()
    nc, ns = info.num_cores, info.num_subcores
    nw = nc * ns
    assert b % (nw * CHUNK) == 0
    b_side = b // nw
    b_per_w = 2 * b_side
    n_chunks = b_per_w // CHUNK
    kpg = dim2 // L
    assert n_chunks % 2 == 0 and n_chunks >= 4

    mesh = plsc.VectorSubcoreMesh(core_axis_name="c", subcore_axis_name="s")
    out_sds = jax.ShapeDtypeStruct((b,), jnp.float32)

    @functools.partial(
        pl.kernel,
        mesh=mesh,
        out_type=(out_sds, out_sds),
        scratch_types=[
            pltpu.VMEM((b_per_w,), jnp.int32),
            pltpu.VMEM((b_per_w,), jnp.int32),
            pltpu.VMEM((b_per_w,), jnp.int32),
            pltpu.VMEM((2, CHUNK, dim2), jnp.float32),
            pltpu.VMEM((2, CHUNK, dim2), jnp.float32),
            pltpu.VMEM((2, CHUNK, dim2), jnp.float32),
            pltpu.VMEM((b_per_w,), jnp.float32),
            pltpu.SemaphoreType.DMA,
            pltpu.SemaphoreType.DMA,
        ],
    )
    def sc_call(pos_h, pos_r, pos_t, neg_h, neg_r, neg_t, ent_hbm, rel_hbm,
                out_pos, out_neg, hidx_v, ridx_v, tidx_v, hbuf, rbuf, tbuf,
                outv, sem_a, sem_b):
        wid = lax.axis_index("s") * nc + lax.axis_index("c")
        base = wid * b_side

        lane = lax.iota(jnp.int32, L)
        swap2d = jnp.reshape(lane ^ 1, (L, 1))
        bfly = [jnp.reshape(lane ^ (1 << p), (L, 1)) for p in range(1, 4)]
        even = (lane & 1) == 0

        src = pl.ds(base, b_side)
        lo, hi = pl.ds(0, b_side), pl.ds(b_side, b_side)
        pltpu.sync_copy(pos_h.at[src], hidx_v.at[lo])
        pltpu.sync_copy(neg_h.at[src], hidx_v.at[hi])
        pltpu.sync_copy(pos_r.at[src], ridx_v.at[lo])
        pltpu.sync_copy(neg_r.at[src], ridx_v.at[hi])
        pltpu.sync_copy(pos_t.at[src], tidx_v.at[lo])
        pltpu.sync_copy(neg_t.at[src], tidx_v.at[hi])

        def copies(ci, slot, sem):
            ids = pl.ds(ci * CHUNK, CHUNK)
            return (
                pltpu.make_async_copy(ent_hbm.at[hidx_v.at[ids]], hbuf.at[slot], sem),
                pltpu.make_async_copy(rel_hbm.at[ridx_v.at[ids]], rbuf.at[slot], sem),
                pltpu.make_async_copy(ent_hbm.at[tidx_v.at[ids]], tbuf.at[slot], sem),
            )

        def start(ci, slot, sem):
            for c in copies(ci, slot, sem):
                c.start()

        def wait(ci, slot, sem):
            for c in copies(ci, slot, sem):
                c.wait()

        def compute(ci, slot):
            def group_body(gi, _):
                e0 = gi * L

                def quad_body(u, svec):
                    eu = e0 + u * EUNROLL
                    for j in range(EUNROLL):
                        e = eu + j
                        acc = jnp.zeros((L,), jnp.float32)
                        for k in range(kpg):
                            h = hbuf[slot, e, pl.ds(k * L, L)]
                            t = tbuf[slot, e, pl.ds(k * L, L)]
                            r = rbuf[slot, e, pl.ds(k * L, L)]
                            ts = _take16(t, swap2d)
                            rs = _take16(r, swap2d)
                            rE = jnp.where(even, r, rs)
                            rOs = jnp.where(even, rs, -r)
                            acc = acc + h * (t * rE + ts * rOs)
                        acc = acc + _take16(acc, swap2d)
                        for p2d in bfly:
                            acc = acc + _take16(acc, p2d)
                        svec = jnp.where(lane == u * EUNROLL + j, acc, svec)
                    return svec

                svec = lax.fori_loop(0, L // EUNROLL, quad_body,
                                     jnp.zeros((L,), jnp.float32))
                outv[pl.ds(ci * CHUNK + e0, L)] = svec
                return 0

            lax.fori_loop(0, CHUNK // L, group_body, 0)

        start(0, 0, sem_a)

        def body(i, _):
            ci_a = 2 * i
            ci_b = ci_a + 1
            start(ci_b, 1, sem_b)
            wait(ci_a, 0, sem_a)
            compute(ci_a, 0)

            @pl.when(i < n_chunks // 2 - 1)
            def _():
                start(ci_a + 2, 0, sem_a)

            wait(ci_b, 1, sem_b)
            compute(ci_b, 1)
            return 0

        lax.fori_loop(0, n_chunks // 2, body, 0)
        pltpu.sync_copy(outv.at[lo], out_pos.at[src])
        pltpu.sync_copy(outv.at[hi], out_neg.at[src])

    return sc_call


def kernel(pos_h, pos_r, pos_t, neg_h, neg_r, neg_t, entity_emb, relation_emb):
    b = pos_h.shape[0]
    dim2 = entity_emb.shape[1]
    args = [pos_h, pos_r, pos_t, neg_h, neg_r, neg_t]
    args = [a if a.dtype == jnp.int32 else a.astype(jnp.int32) for a in args]
    sc_call = _make_sc_call(b, dim2)
    return sc_call(*args, entity_emb, relation_emb)


# speedup vs baseline: 2.0091x; 2.0091x over previous
import functools

import jax
import jax.numpy as jnp
from jax import lax
from jax.experimental import pallas as pl
from jax.experimental.pallas import tpu as pltpu
from jax.experimental.pallas import tpu_sc as plsc

L = 16
CHUNK = 128
EUNROLL = 4

_GATHER_DNUMS = lax.GatherDimensionNumbers(
    offset_dims=(), collapsed_slice_dims=(0,), start_index_map=(0,))


def _take16(x, idx2d):
    return lax.gather(x, idx2d, _GATHER_DNUMS, (1,),
                      mode=lax.GatherScatterMode.PROMISE_IN_BOUNDS)


def _make_sc_call(b, dim2):
    info = plsc.get_sparse_core_info()
    nc, ns = info.num_cores, info.num_subcores
    nw = nc * ns
    assert b % (nw * CHUNK) == 0
    b_side = b // nw
    b_per_w = 2 * b_side
    n_chunks = b_per_w // CHUNK
    kpg = dim2 // L
    assert n_chunks % 2 == 0 and n_chunks >= 4

    mesh = plsc.VectorSubcoreMesh(core_axis_name="c", subcore_axis_name="s")
    out_sds = jax.ShapeDtypeStruct((b,), jnp.float32)

    @functools.partial(
        pl.kernel,
        mesh=mesh,
        out_type=(out_sds, out_sds),
        scratch_types=[
            pltpu.VMEM((b_per_w,), jnp.int32),
            pltpu.VMEM((b_per_w,), jnp.int32),
            pltpu.VMEM((b_per_w,), jnp.int32),
            pltpu.VMEM((2, CHUNK, dim2), jnp.float32),
            pltpu.VMEM((2, CHUNK, dim2), jnp.float32),
            pltpu.VMEM((2, CHUNK, dim2), jnp.float32),
            pltpu.VMEM((b_per_w,), jnp.float32),
            pltpu.SemaphoreType.DMA,
            pltpu.SemaphoreType.DMA,
        ],
    )
    def sc_call(pos_h, pos_r, pos_t, neg_h, neg_r, neg_t, ent_hbm, rel_hbm,
                out_pos, out_neg, hidx_v, ridx_v, tidx_v, hbuf, rbuf, tbuf,
                outv, sem_a, sem_b):
        wid = lax.axis_index("s") * nc + lax.axis_index("c")
        base = wid * b_side

        lane = lax.iota(jnp.int32, L)
        swap2d = jnp.reshape(lane ^ 1, (L, 1))
        bfly = [jnp.reshape(lane ^ (1 << p), (L, 1)) for p in range(1, 4)]
        even = (lane & 1) == 0

        src = pl.ds(base, b_side)
        lo, hi = pl.ds(0, b_side), pl.ds(b_side, b_side)
        pltpu.sync_copy(pos_h.at[src], hidx_v.at[lo])
        pltpu.sync_copy(neg_h.at[src], hidx_v.at[hi])
        pltpu.sync_copy(pos_r.at[src], ridx_v.at[lo])
        pltpu.sync_copy(neg_r.at[src], ridx_v.at[hi])
        pltpu.sync_copy(pos_t.at[src], tidx_v.at[lo])
        pltpu.sync_copy(neg_t.at[src], tidx_v.at[hi])

        def copies(ci, slot, sem):
            ids = pl.ds(ci * CHUNK, CHUNK)
            return (
                pltpu.make_async_copy(ent_hbm.at[hidx_v.at[ids]], hbuf.at[slot], sem),
                pltpu.make_async_copy(rel_hbm.at[ridx_v.at[ids]], rbuf.at[slot], sem),
                pltpu.make_async_copy(ent_hbm.at[tidx_v.at[ids]], tbuf.at[slot], sem),
            )

        def start(ci, slot, sem):
            for c in copies(ci, slot, sem):
                c.start()

        def wait(ci, slot, sem):
            for c in copies(ci, slot, sem):
                c.wait()

        def compute(ci, slot):
            def group_body(gi, _):
                e0 = gi * L
                svec = jnp.zeros((L,), jnp.float32)
                for e16 in range(L):
                    e = e0 + e16
                    acc = jnp.zeros((L,), jnp.float32)
                    for k in range(kpg):
                        h = hbuf[slot, e, pl.ds(k * L, L)]
                        t = tbuf[slot, e, pl.ds(k * L, L)]
                        r = rbuf[slot, e, pl.ds(k * L, L)]
                        ts = _take16(t, swap2d)
                        rs = _take16(r, swap2d)
                        rE = jnp.where(even, r, rs)
                        rOs = jnp.where(even, rs, -r)
                        acc = acc + h * (t * rE + ts * rOs)
                    acc = acc + _take16(acc, swap2d)
                    for p2d in bfly:
                        acc = acc + _take16(acc, p2d)
                    svec = jnp.where(lane == e16, acc, svec)
                outv[pl.ds(ci * CHUNK + e0, L)] = svec
                return 0

            lax.fori_loop(0, CHUNK // L, group_body, 0)

        start(0, 0, sem_a)

        def body(i, _):
            ci_a = 2 * i
            ci_b = ci_a + 1
            start(ci_b, 1, sem_b)
            wait(ci_a, 0, sem_a)
            compute(ci_a, 0)

            @pl.when(i < n_chunks // 2 - 1)
            def _():
                start(ci_a + 2, 0, sem_a)

            wait(ci_b, 1, sem_b)
            compute(ci_b, 1)
            return 0

        lax.fori_loop(0, n_chunks // 2, body, 0)
        pltpu.sync_copy(outv.at[lo], out_pos.at[src])
        pltpu.sync_copy(outv.at[hi], out_neg.at[src])

    return sc_call


def kernel(pos_h, pos_r, pos_t, neg_h, neg_r, neg_t, entity_emb, relation_emb):
    b = pos_h.shape[0]
    dim2 = entity_emb.shape[1]
    args = [pos_h, pos_r, pos_t, neg_h, neg_r, neg_t]
    args = [a if a.dtype == jnp.int32 else a.astype(jnp.int32) for a in args]
    sc_call = _make_sc_call(b, dim2)
    return sc_call(*args, entity_emb, relation_emb)


# speedup vs baseline: 2.0433x; 1.0170x over previous
import functools

import jax
import jax.numpy as jnp
from jax import lax
from jax.experimental import pallas as pl
from jax.experimental.pallas import tpu as pltpu
from jax.experimental.pallas import tpu_sc as plsc

L = 16
CHUNK = 128

_GATHER_DNUMS = lax.GatherDimensionNumbers(
    offset_dims=(), collapsed_slice_dims=(0,), start_index_map=(0,))


def _take16(x, idx2d):
    return lax.gather(x, idx2d, _GATHER_DNUMS, (1,),
                      mode=lax.GatherScatterMode.PROMISE_IN_BOUNDS)


def _make_sc_call(b, dim2):
    info = plsc.get_sparse_core_info()
    nc, ns = info.num_cores, info.num_subcores
    nw = nc * ns
    assert b % (nw * CHUNK) == 0
    b_side = b // nw
    b_per_w = 2 * b_side
    n_chunks = b_per_w // CHUNK
    kpg = dim2 // L
    assert n_chunks % 2 == 0 and n_chunks >= 4

    mesh = plsc.VectorSubcoreMesh(core_axis_name="c", subcore_axis_name="s")
    out_sds = jax.ShapeDtypeStruct((b,), jnp.float32)

    @functools.partial(
        pl.kernel,
        mesh=mesh,
        out_type=(out_sds, out_sds),
        scratch_types=[
            pltpu.VMEM((b_per_w,), jnp.int32),
            pltpu.VMEM((b_per_w,), jnp.int32),
            pltpu.VMEM((b_per_w,), jnp.int32),
            pltpu.VMEM((2, CHUNK, dim2), jnp.float32),
            pltpu.VMEM((2, CHUNK, dim2), jnp.float32),
            pltpu.VMEM((2, CHUNK, dim2), jnp.float32),
            pltpu.VMEM((b_per_w,), jnp.float32),
            pltpu.SemaphoreType.DMA,
            pltpu.SemaphoreType.DMA,
        ],
    )
    def sc_call(pos_h, pos_r, pos_t, neg_h, neg_r, neg_t, ent_hbm, rel_hbm,
                out_pos, out_neg, hidx_v, ridx_v, tidx_v, hbuf, rbuf, tbuf,
                outv, sem_a, sem_b):
        wid = lax.axis_index("s") * nc + lax.axis_index("c")
        base = wid * b_side

        lane = lax.iota(jnp.int32, L)
        swap2d = jnp.reshape(lane ^ 1, (L, 1))
        bfly = [jnp.reshape(lane ^ (1 << p), (L, 1)) for p in range(1, 4)]
        even = (lane & 1) == 0

        src = pl.ds(base, b_side)
        lo, hi = pl.ds(0, b_side), pl.ds(b_side, b_side)
        pltpu.sync_copy(pos_h.at[src], hidx_v.at[lo])
        pltpu.sync_copy(neg_h.at[src], hidx_v.at[hi])
        pltpu.sync_copy(pos_r.at[src], ridx_v.at[lo])
        pltpu.sync_copy(neg_r.at[src], ridx_v.at[hi])
        pltpu.sync_copy(pos_t.at[src], tidx_v.at[lo])
        pltpu.sync_copy(neg_t.at[src], tidx_v.at[hi])

        def copies(ci, slot, sem):
            ids = pl.ds(ci * CHUNK, CHUNK)
            return (
                pltpu.make_async_copy(ent_hbm.at[hidx_v.at[ids]], hbuf.at[slot], sem),
                pltpu.make_async_copy(rel_hbm.at[ridx_v.at[ids]], rbuf.at[slot], sem),
                pltpu.make_async_copy(ent_hbm.at[tidx_v.at[ids]], tbuf.at[slot], sem),
            )

        def start(ci, slot, sem):
            for c in copies(ci, slot, sem):
                c.start()

        def wait(ci, slot, sem):
            for c in copies(ci, slot, sem):
                c.wait()

        def compute(ci, slot):
            def group_body(gi, _):
                e0 = gi * L
                svec = jnp.zeros((L,), jnp.float32)
                for e16 in range(L):
                    e = e0 + e16
                    acc = jnp.zeros((L,), jnp.float32)
                    for k in range(kpg):
                        h = hbuf[slot, e, pl.ds(k * L, L)]
                        t = tbuf[slot, e, pl.ds(k * L, L)]
                        r = rbuf[slot, e, pl.ds(k * L, L)]
                        ts = _take16(t, swap2d)
                        rs = _take16(r, swap2d)
                        rE = jnp.where(even, r, rs)
                        rOs = jnp.where(even, rs, -r)
                        acc = acc + h * (t * rE + ts * rOs)
                    acc = acc + _take16(acc, swap2d)
                    for p2d in bfly:
                        acc = acc + _take16(acc, p2d)
                    svec = jnp.where(lane == e16, acc, svec)
                outv[pl.ds(ci * CHUNK + e0, L)] = svec
                return 0

            lax.fori_loop(0, CHUNK // L, group_body, 0)

        start(0, 0, sem_a)

        def body(ci, _):
            slot = ci & 1

            @pl.when(ci + 1 < n_chunks)
            def _():
                @pl.when(slot == 0)
                def _():
                    start(ci + 1, 1, sem_b)

                @pl.when(slot == 1)
                def _():
                    start(ci + 1, 0, sem_a)

            @pl.when(slot == 0)
            def _():
                wait(ci, 0, sem_a)

            @pl.when(slot == 1)
            def _():
                wait(ci, 1, sem_b)

            compute(ci, slot)
            return 0

        lax.fori_loop(0, n_chunks, body, 0)
        pltpu.sync_copy(outv.at[lo], out_pos.at[src])
        pltpu.sync_copy(outv.at[hi], out_neg.at[src])

    return sc_call


def kernel(pos_h, pos_r, pos_t, neg_h, neg_r, neg_t, entity_emb, relation_emb):
    b = pos_h.shape[0]
    dim2 = entity_emb.shape[1]
    args = [pos_h, pos_r, pos_t, neg_h, neg_r, neg_t]
    args = [a if a.dtype == jnp.int32 else a.astype(jnp.int32) for a in args]
    sc_call = _make_sc_call(b, dim2)
    return sc_call(*args, entity_emb, relation_emb)


# speedup vs baseline: 2.1354x; 1.0451x over previous
import functools

import jax
import jax.numpy as jnp
from jax import lax
from jax.experimental import pallas as pl
from jax.experimental.pallas import tpu as pltpu
from jax.experimental.pallas import tpu_sc as plsc

L = 16
CHUNK = 128

_GATHER_DNUMS = lax.GatherDimensionNumbers(
    offset_dims=(), collapsed_slice_dims=(0,), start_index_map=(0,))


def _take16(x, idx2d):
    return lax.gather(x, idx2d, _GATHER_DNUMS, (1,),
                      mode=lax.GatherScatterMode.PROMISE_IN_BOUNDS)


def _make_sc_call(b, dim2):
    info = plsc.get_sparse_core_info()
    nc, ns = info.num_cores, info.num_subcores
    nw = nc * ns
    assert b % (nw * CHUNK) == 0
    b_side = b // nw
    b_per_w = 2 * b_side
    n_chunks = b_per_w // CHUNK
    kpg = dim2 // L
    assert n_chunks % 2 == 0 and n_chunks >= 4

    mesh = plsc.VectorSubcoreMesh(core_axis_name="c", subcore_axis_name="s")
    out_sds = jax.ShapeDtypeStruct((b,), jnp.float32)

    @functools.partial(
        pl.kernel,
        mesh=mesh,
        out_type=(out_sds, out_sds),
        scratch_types=[
            pltpu.VMEM((b_per_w,), jnp.int32),
            pltpu.VMEM((b_per_w,), jnp.int32),
            pltpu.VMEM((b_per_w,), jnp.int32),
            pltpu.VMEM((2, CHUNK, dim2), jnp.float32),
            pltpu.VMEM((2, CHUNK, dim2), jnp.float32),
            pltpu.VMEM((2, CHUNK, dim2), jnp.float32),
            pltpu.VMEM((b_per_w,), jnp.float32),
            pltpu.SemaphoreType.DMA,
            pltpu.SemaphoreType.DMA,
        ],
    )
    def sc_call(pos_h, pos_r, pos_t, neg_h, neg_r, neg_t, ent_hbm, rel_hbm,
                out_pos, out_neg, hidx_v, ridx_v, tidx_v, hbuf, rbuf, tbuf,
                outv, sem_a, sem_b):
        wid = lax.axis_index("s") * nc + lax.axis_index("c")
        base = wid * b_side

        lane = lax.iota(jnp.int32, L)
        swap2d = jnp.reshape(lane ^ 1, (L, 1))
        bfly = [jnp.reshape(lane ^ (1 << p), (L, 1)) for p in range(1, 4)]
        even = (lane & 1) == 0

        src = pl.ds(base, b_side)
        lo, hi = pl.ds(0, b_side), pl.ds(b_side, b_side)
        idx_copies = [
            pltpu.make_async_copy(pos_h.at[src], hidx_v.at[lo], sem_a),
            pltpu.make_async_copy(neg_h.at[src], hidx_v.at[hi], sem_a),
            pltpu.make_async_copy(pos_r.at[src], ridx_v.at[lo], sem_a),
            pltpu.make_async_copy(neg_r.at[src], ridx_v.at[hi], sem_a),
            pltpu.make_async_copy(pos_t.at[src], tidx_v.at[lo], sem_a),
            pltpu.make_async_copy(neg_t.at[src], tidx_v.at[hi], sem_a),
        ]
        for c in idx_copies:
            c.start()
        for c in idx_copies:
            c.wait()

        def copies(ci, slot, sem):
            ids = pl.ds(ci * CHUNK, CHUNK)
            return (
                pltpu.make_async_copy(ent_hbm.at[hidx_v.at[ids]], hbuf.at[slot], sem),
                pltpu.make_async_copy(rel_hbm.at[ridx_v.at[ids]], rbuf.at[slot], sem),
                pltpu.make_async_copy(ent_hbm.at[tidx_v.at[ids]], tbuf.at[slot], sem),
            )

        def start(ci, slot, sem):
            for c in copies(ci, slot, sem):
                c.start()

        def wait(ci, slot, sem):
            for c in copies(ci, slot, sem):
                c.wait()

        def compute(ci, slot):
            def group_body(gi, _):
                e0 = gi * L
                svec = jnp.zeros((L,), jnp.float32)
                for e16 in range(L):
                    e = e0 + e16
                    acc = jnp.zeros((L,), jnp.float32)
                    for kk in range(kpg // 2):
                        for half in range(2):
                            k = 2 * kk + half
                            h = hbuf[slot, e, pl.ds(k * L, L)]
                            t = tbuf[slot, e, pl.ds(k * L, L)]
                            r = rbuf[slot, e, pl.ds(k * L, L)]
                            ts = _take16(t, swap2d)
                            rs = _take16(r, swap2d)
                            rE = jnp.where(even, r, rs)
                            rOs = jnp.where(even, rs, -r)
                            acc = acc + h * (t * rE + ts * rOs)
                    acc = acc + _take16(acc, swap2d)
                    for p2d in bfly:
                        acc = acc + _take16(acc, p2d)
                    svec = jnp.where(lane == e16, acc, svec)
                outv[pl.ds(ci * CHUNK + e0, L)] = svec
                return 0

            lax.fori_loop(0, CHUNK // L, group_body, 0)

        start(0, 0, sem_a)

        def body(ci, _):
            slot = ci & 1

            @pl.when(ci + 1 < n_chunks)
            def _():
                @pl.when(slot == 0)
                def _():
                    start(ci + 1, 1, sem_b)

                @pl.when(slot == 1)
                def _():
                    start(ci + 1, 0, sem_a)

            @pl.when(slot == 0)
            def _():
                wait(ci, 0, sem_a)

            @pl.when(slot == 1)
            def _():
                wait(ci, 1, sem_b)

            compute(ci, slot)
            return 0

        lax.fori_loop(0, n_chunks, body, 0)
        pltpu.sync_copy(outv.at[lo], out_pos.at[src])
        pltpu.sync_copy(outv.at[hi], out_neg.at[src])

    return sc_call


def kernel(pos_h, pos_r, pos_t, neg_h, neg_r, neg_t, entity_emb, relation_emb):
    b = pos_h.shape[0]
    dim2 = entity_emb.shape[1]
    args = [pos_h, pos_r, pos_t, neg_h, neg_r, neg_t]
    args = [a if a.dtype == jnp.int32 else a.astype(jnp.int32) for a in args]
    sc_call = _make_sc_call(b, dim2)
    return sc_call(*args, entity_emb, relation_emb)
